# baseline (speedup 1.0000x reference)
import jax
import jax.numpy as jnp
from jax.experimental import pallas as pl
from jax.experimental.pallas import tpu as pltpu

_N = 8192
_D = 2048
_E = 16
_TAU = 1.0
_LAM = 0.04
_BLK = 1024
_NBLK = _N // _BLK


def _softmax0(z):
    z = z - jnp.max(z, axis=0, keepdims=True)
    e = jnp.exp(z)
    return e / jnp.sum(e, axis=0, keepdims=True)


def _mm(a, b, ca, cb):
    return jax.lax.dot_general(a, b, (((ca,), (cb,)), ((), ())),
                               preferred_element_type=jnp.float32)


def _router_kernel(x_ref, w_ref, b_ref, m_ref, mask_ref, ct_ref, m1_ref, c_ref):
    i = pl.program_id(0)

    logits_t = _mm(w_ref[...], x_ref[...], 1, 1) + b_ref[...]
    centered_t = logits_t - jnp.mean(logits_t, axis=0, keepdims=True)
    m1_t = _softmax0(centered_t / _TAU)

    @pl.when(i == 0)
    def _():
        c_ref[...] = jnp.zeros_like(c_ref)

    c_ref[...] += _mm(m1_t, m1_t, 1, 1)
    ct_ref[:, pl.ds(i * _BLK, _BLK)] = centered_t
    m1_ref[:, pl.ds(i * _BLK, _BLK)] = m1_t

    @pl.when(i == _NBLK - 1)
    def _():
        cen = ct_ref[...]
        m1 = m1_ref[...]
        ri = jax.lax.broadcasted_iota(jnp.int32, (_E, _E), 0)
        ci = jax.lax.broadcasted_iota(jnp.int32, (_E, _E), 1)
        c_od = jnp.where(ri == ci, 0.0, c_ref[...])
        grad_t = 4.0 * _mm(c_od, m1, 0, 0)
        t = m1 * grad_t
        exact_grad = t - m1 * jnp.sum(t, axis=0, keepdims=True)
        ez = jnp.exp((cen - _LAM * exact_grad) / _TAU)
        m = ez / jnp.sum(ez, axis=0, keepdims=True)
        for _ in range(10):
            col = jnp.sum(m, axis=1, keepdims=True)
            m = m * ((_E / _N) / jnp.maximum(col, 1e-12))
            row = jnp.sum(m, axis=0, keepdims=True)
            m = m / jnp.maximum(row, 1e-12)

        eidx = jax.lax.broadcasted_iota(jnp.int32, (_E, _N), 0)
        a1 = jnp.argmax(m, axis=0)[None, :]
        hit1 = eidx == a1
        m2 = jnp.where(hit1, -jnp.inf, m)
        a2 = jnp.argmax(m2, axis=0)[None, :]
        mask_f = jnp.where(hit1 | (eidx == a2), 1.0, 0.0)
        m_ref[...] = m.T
        mask_ref[...] = mask_f.T


def kernel(x, W, b):
    m, mask = pl.pallas_call(
        _router_kernel,
        grid=(_NBLK,),
        in_specs=[
            pl.BlockSpec((_BLK, _D), lambda i: (i, 0)),
            pl.BlockSpec((_E, _D), lambda i: (0, 0)),
            pl.BlockSpec((_E, 1), lambda i: (0, 0)),
        ],
        out_specs=[
            pl.BlockSpec((_N, _E), lambda i: (0, 0)),
            pl.BlockSpec((_N, _E), lambda i: (0, 0)),
        ],
        out_shape=[
            jax.ShapeDtypeStruct((_N, _E), jnp.float32),
            jax.ShapeDtypeStruct((_N, _E), jnp.float32),
        ],
        scratch_shapes=[
            pltpu.VMEM((_E, _N), jnp.float32),
            pltpu.VMEM((_E, _N), jnp.float32),
            pltpu.VMEM((_E, _E), jnp.float32),
        ],
    )(x, W, b.reshape(_E, 1))
    return (m, mask.astype(jnp.bool_))


# speedup vs baseline: 1.0010x; 1.0010x over previous
import jax
import jax.numpy as jnp
from jax.experimental import pallas as pl
from jax.experimental.pallas import tpu as pltpu

_N = 8192
_D = 2048
_E = 16
_TAU = 1.0
_LAM = 0.04
_BLK = 1024
_NBLK = _N // _BLK


def _softmax0(z):
    z = z - jnp.max(z, axis=0, keepdims=True)
    e = jnp.exp(z)
    return e / jnp.sum(e, axis=0, keepdims=True)


def _mm(a, b, ca, cb):
    return jax.lax.dot_general(a, b, (((ca,), (cb,)), ((), ())),
                               preferred_element_type=jnp.float32)


def _router_kernel(x_ref, w_ref, b_ref, m_ref, mask_ref, ct_ref, m1_ref, c_ref):
    i = pl.program_id(0)

    logits_t = _mm(w_ref[...], x_ref[...], 1, 1) + b_ref[...]
    centered_t = logits_t - jnp.mean(logits_t, axis=0, keepdims=True)
    m1_t = _softmax0(centered_t / _TAU)

    @pl.when(i == 0)
    def _():
        c_ref[...] = jnp.zeros_like(c_ref)

    c_ref[...] += _mm(m1_t, m1_t, 1, 1)
    ct_ref[:, pl.ds(i * _BLK, _BLK)] = centered_t
    m1_ref[:, pl.ds(i * _BLK, _BLK)] = m1_t

    @pl.when(i == _NBLK - 1)
    def _():
        cen = ct_ref[...]
        m1 = m1_ref[...]
        ri = jax.lax.broadcasted_iota(jnp.int32, (_E, _E), 0)
        ci = jax.lax.broadcasted_iota(jnp.int32, (_E, _E), 1)
        c_od = jnp.where(ri == ci, 0.0, c_ref[...])
        grad_t = 4.0 * _mm(c_od, m1, 0, 0)
        t = m1 * grad_t
        exact_grad = t - m1 * jnp.sum(t, axis=0, keepdims=True)
        ez = jnp.exp((cen - _LAM * exact_grad) / _TAU)
        m = ez / jnp.sum(ez, axis=0, keepdims=True)
        for _ in range(10):
            col = jnp.sum(m, axis=1, keepdims=True)
            m = m * ((_E / _N) / jnp.maximum(col, 1e-12))
            row = jnp.sum(m, axis=0, keepdims=True)
            m = m / jnp.maximum(row, 1e-12)

        eidx = jax.lax.broadcasted_iota(jnp.int32, (_E, _N), 0)
        a1 = jnp.argmax(m, axis=0)[None, :]
        hit1 = eidx == a1
        m2 = jnp.where(hit1, -jnp.inf, m)
        a2 = jnp.argmax(m2, axis=0)[None, :]
        mask_f = jnp.where(hit1 | (eidx == a2), 1.0, 0.0)
        m_ref[...] = m.T
        mask_ref[...] = mask_f.T > 0.5


def kernel(x, W, b):
    m, mask = pl.pallas_call(
        _router_kernel,
        grid=(_NBLK,),
        in_specs=[
            pl.BlockSpec((_BLK, _D), lambda i: (i, 0)),
            pl.BlockSpec((_E, _D), lambda i: (0, 0)),
            pl.BlockSpec((_E, 1), lambda i: (0, 0)),
        ],
        out_specs=[
            pl.BlockSpec((_N, _E), lambda i: (0, 0)),
            pl.BlockSpec((_N, _E), lambda i: (0, 0)),
        ],
        out_shape=[
            jax.ShapeDtypeStruct((_N, _E), jnp.float32),
            jax.ShapeDtypeStruct((_N, _E), jnp.bool_),
        ],
        scratch_shapes=[
            pltpu.VMEM((_E, _N), jnp.float32),
            pltpu.VMEM((_E, _N), jnp.float32),
            pltpu.VMEM((_E, _E), jnp.float32),
        ],
    )(x, W, b.reshape(_E, 1))
    return (m, mask)
